# baseline (speedup 1.0000x reference)
import functools

import jax
import jax.numpy as jnp
from jax import lax
from jax.experimental import pallas as pl
from jax.experimental.pallas import tpu as pltpu
from jax.experimental.pallas import tpu_sc as plsc

B = 16384
C = 64
NC = 2
NS = 16
NW = NC * NS
B_SC = 10240
B_TC = B - B_SC
BPW = B_SC // NW
L = 16
NG = BPW // L
NBUF = 2
RPS = 8
TSTEPS = B_TC // RPS


def _vsqrt(x):
    i = lax.bitcast_convert_type(x, jnp.int32)
    i = (i >> 1) + jnp.int32(0x1FBD1DF5)
    y = lax.bitcast_convert_type(i, jnp.float32)
    y = 0.5 * (y + x / y)
    y = 0.5 * (y + x / y)
    y = 0.5 * (y + x / y)
    return y


def _lanesum(v):
    acc = v[0]
    for i in range(1, L):
        acc = acc + v[i]
    return acc


def _body(user_hbm, item_hbm, ustruct_hbm, istruct_hbm, rh_hbm, rel_hbm,
          out_hbm, uidx_v, iidx_v, ublk_v, iblk_v, urow_v, irow_v, rh_v,
          rel_v, out_v, sem0, sem1):
    wid = lax.axis_index("s") * NC + lax.axis_index("c")
    base = wid * BPW

    pltpu.sync_copy(user_hbm.at[pl.ds(base, BPW)], uidx_v)
    pltpu.sync_copy(item_hbm.at[pl.ds(base, BPW)], iidx_v)
    pltpu.sync_copy(rh_hbm, rh_v)
    pltpu.sync_copy(rel_hbm, rel_v)

    sems = [sem0, sem1]

    rh = [rh_v[pl.ds(k * L, L)] for k in range(C // L)]
    rele = [rel_v[pl.ds(k * L, L)] + 1e-6 for k in range(C // L)]
    s = rh[0] * rh[0]
    p = rh[0] * rele[0]
    for k in range(1, C // L):
        s = s + rh[k] * rh[k]
        p = p + rh[k] * rele[k]
    n2 = _lanesum(s)
    n2v = jnp.full((L,), 0.0, jnp.float32) + n2
    invv = 1.0 / jnp.maximum(_vsqrt(n2v), 1e-12)
    inv = invv[0]
    g2 = n2 * inv * inv
    rho = _lanesum(p) * inv
    ca = 2.0 - g2
    cb = 2.0 * rho
    rhn = [r * inv for r in rh]

    iota = lax.iota(jnp.int32, L)
    zero = jnp.zeros((L,), jnp.int32)

    def issue_pair(uvecs, ivecs, k0, slot):
        for h in range(2):
            offu = pl.multiple_of((uvecs[k0 + h] >> 7) * 128, 128)
            offi = pl.multiple_of((ivecs[k0 + h] >> 7) * 128, 128)
            hs = pl.ds(h * 128, 128)
            pltpu.async_copy(ustruct_hbm.at[:, pl.ds(offu, 128)],
                             ublk_v.at[slot].at[:, hs], sems[slot])
            pltpu.async_copy(istruct_hbm.at[:, pl.ds(offi, 128)],
                             iblk_v.at[slot].at[:, hs], sems[slot])

    def drain(slot):
        pltpu.make_async_copy(ustruct_hbm.at[:, pl.ds(0, 256)],
                              ublk_v.at[slot], sems[slot]).wait()
        pltpu.make_async_copy(istruct_hbm.at[:, pl.ds(0, 256)],
                              iblk_v.at[slot], sems[slot]).wait()

    uvec0 = uidx_v[pl.ds(0, L)]
    ivec0 = iidx_v[pl.ds(0, L)]
    issue_pair(uvec0, ivec0, 0, 0)
    issue_pair(uvec0, ivec0, 2, 1)

    def group_body(g, _):
        gsl = pl.ds(g * L, L)
        uvec = uidx_v[gsl]
        ivec = iidx_v[gsl]
        nsl = pl.ds(((g + 1) % NG) * L, L)
        uvn = uidx_v[nsl]
        ivn = iidx_v[nsl]

        def extract(k, slot, half):
            lu = (zero + (uvec[k] & 127)) + half * 128
            li = (zero + (ivec[k] & 127)) + half * 128
            for k4 in range(C // L):
                fv = k4 * L + iota
                urow_v[k, pl.ds(k4 * L, L)] = plsc.load_gather(
                    ublk_v.at[slot], [fv, lu])
                irow_v[k, pl.ds(k4 * L, L)] = plsc.load_gather(
                    iblk_v.at[slot], [fv, li])

        for m in range(L // 2):
            ss = m % 2
            drain(ss)
            extract(2 * m, ss, 0)
            extract(2 * m + 1, ss, 1)
            if 2 * m + 4 < L:
                issue_pair(uvec, ivec, 2 * m + 4, ss)
            else:
                issue_pair(uvn, ivn, 2 * m + 4 - L, ss)

        acc_a = jnp.zeros((L,), jnp.float32)
        acc_d = jnp.zeros((L,), jnp.float32)
        for c in range(C):
            cvec = zero + c
            u = plsc.load_gather(urow_v, [iota, cvec])
            it = plsc.load_gather(irow_v, [iota, cvec])
            d = u - it
            e = d + rele[c // L][c % L]
            acc_a = acc_a + e * e
            acc_d = acc_d + d * rhn[c // L][c % L]
        ssq = acc_a - ca * acc_d * acc_d - cb * acc_d
        out_v[gsl] = _vsqrt(ssq)
        return 0

    lax.fori_loop(0, NG, group_body, 0)

    drain(0)
    drain(1)

    pltpu.sync_copy(out_v, out_hbm.at[pl.ds(base, BPW)])


@jax.jit
def _transh(user, item, user_structure_t, item_structure_t, rh, rel):
    mesh = plsc.VectorSubcoreMesh(core_axis_name="c", subcore_axis_name="s")
    return pl.kernel(
        _body,
        out_type=jax.ShapeDtypeStruct((B_SC,), jnp.float32),
        mesh=mesh,
        compiler_params=pltpu.CompilerParams(needs_layout_passes=False),
        scratch_types=[
            pltpu.VMEM((BPW,), jnp.int32),
            pltpu.VMEM((BPW,), jnp.int32),
            pltpu.VMEM((NBUF, C, 256), jnp.float32),
            pltpu.VMEM((NBUF, C, 256), jnp.float32),
            pltpu.VMEM((L, C), jnp.float32),
            pltpu.VMEM((L, C), jnp.float32),
            pltpu.VMEM((C,), jnp.float32),
            pltpu.VMEM((C,), jnp.float32),
            pltpu.VMEM((BPW,), jnp.float32),
            pltpu.SemaphoreType.DMA,
            pltpu.SemaphoreType.DMA,
        ],
    )(user, item, user_structure_t, item_structure_t, rh, rel)


def _tc_body(ublk_s, ulane_s, iblk_s, ilane_s, rh_ref, rel_ref, *refs):
    ubs = refs[:RPS]
    ibs = refs[RPS:2 * RPS]
    out_ref = refs[2 * RPS]
    i = pl.program_id(0)

    rh2 = rh_ref[...]
    rele2 = rel_ref[...] + 1e-6
    n2 = jnp.sum(rh2 * rh2)
    inv = 1.0 / jnp.maximum(jnp.sqrt(n2), 1e-12)
    rhn2 = rh2 * inv
    rho = jnp.sum(rhn2 * rele2)
    ca = 2.0 - n2 * inv * inv
    cb = 2.0 * rho

    p_iota = lax.broadcasted_iota(jnp.int32, (RPS, RPS * 128), 1)
    r_iota = lax.broadcasted_iota(jnp.int32, (RPS, RPS * 128), 0)

    def extract(blocks, lane_s):
        lanes = jnp.zeros((RPS, RPS * 128), jnp.int32)
        for r in range(RPS):
            lanes = jnp.where(r_iota == r, lane_s[i * RPS + r], lanes)
        onehot = jnp.where(((p_iota >> 7) == r_iota)
                           & ((p_iota & 127) == lanes), 1.0, 0.0)
        cat = jnp.concatenate([b[...] for b in blocks], axis=1)
        return lax.dot_general(onehot, cat, (((1,), (1,)), ((), ())),
                               precision=lax.Precision.HIGHEST)

    U = extract(ubs, ulane_s)
    I = extract(ibs, ilane_s)
    D = U - I
    E = D + rele2
    a = jnp.sum(E * E, axis=1)
    dot = jnp.sum(D * rhn2, axis=1)
    ssq = a - ca * dot * dot - cb * dot
    out_ref[...] = jnp.sqrt(ssq).reshape(1, 1, RPS)


@jax.jit
def _transh_tc(ublk, ulane, iblk, ilane, user_structure_t, item_structure_t,
               rh2, rel2):
    uspec = [pl.BlockSpec(
        (C, 128), lambda i, ub, ul, ib, il, r=r: (0, ub[i * RPS + r]))
        for r in range(RPS)]
    ispec = [pl.BlockSpec(
        (C, 128), lambda i, ub, ul, ib, il, r=r: (0, ib[i * RPS + r]))
        for r in range(RPS)]
    grid_spec = pltpu.PrefetchScalarGridSpec(
        num_scalar_prefetch=4,
        grid=(TSTEPS,),
        in_specs=[
            pl.BlockSpec((1, C), lambda i, *_: (0, 0)),
            pl.BlockSpec((1, C), lambda i, *_: (0, 0)),
            *uspec,
            *ispec,
        ],
        out_specs=pl.BlockSpec((1, 1, RPS), lambda i, *_: (i, 0, 0)),
    )
    out = pl.pallas_call(
        _tc_body,
        grid_spec=grid_spec,
        out_shape=jax.ShapeDtypeStruct((TSTEPS, 1, RPS), jnp.float32),
    )(ublk, ulane, iblk, ilane, rh2, rel2,
      *([user_structure_t] * RPS), *([item_structure_t] * RPS))
    return out.reshape(B_TC)


@jax.jit
def _combined(user, item, user_structure, item_structure, rh2, rel2):
    ust = user_structure.T
    ist = item_structure.T
    sc_out = _transh(user[:B_SC], item[:B_SC], ust, ist,
                     rh2.reshape(C), rel2.reshape(C))
    ut = user[B_SC:]
    it = item[B_SC:]
    tc_out = _transh_tc(ut >> 7, ut & 127, it >> 7, it & 127, ust, ist,
                        rh2, rel2)
    return jnp.concatenate([sc_out, tc_out])


def kernel(user, item, user_structure, item_structure, relation_embedding,
           relationHyper):
    return _combined(user.astype(jnp.int32), item.astype(jnp.int32),
                     user_structure, item_structure, relationHyper,
                     relation_embedding)
